# Initial kernel scaffold; baseline (speedup 1.0000x reference)
import jax
import jax.numpy as jnp
from jax.experimental import pallas as pl


def kernel(X, edge_index, edge_weight, W1, b1, W2, b2):
    raise NotImplementedError("write your pallas kernel here")


# speedup vs baseline: 2.9323x; 2.9323x over previous
import functools

import jax
import jax.numpy as jnp
from jax import lax
from jax.experimental import pallas as pl
from jax.experimental.pallas import tpu as pltpu
from jax.experimental.pallas import tpu_sc as plsc

N_NODES = 10000
N_EDGES = 160000
E_PAD = 163840
IN_FEATS = 256
H_FEATS = 512
NUM_CLASSES = 128

NC = 2
NS = 16
LANES = 16
C = 128


def _spmm_body(table, src, dst, w, out, acc, srcb, gixb, dstb, wb, rows, zrow,
               sem, *, fsplit, EB, ZR):
  c = lax.axis_index("c")
  s = lax.axis_index("s")
  big = 640
  last = N_NODES - big * (NS - 1)
  row0 = s * big
  nwork = NS if fsplit else NC * NS
  ne = E_PAD // nwork
  nchunk = ne // EB

  def _zr_body(r, _):
    for k in range(C // LANES):
      zrow[r, pl.ds(k * LANES, LANES)] = jnp.zeros((LANES,), jnp.float32)
    return _
  lax.fori_loop(0, ZR, _zr_body, None)

  nzblk = jnp.where(s == NS - 1, last // ZR, big // ZR)

  def _zcopy(j, _):
    pltpu.sync_copy(zrow, acc.at[pl.ds(row0 + j * ZR, ZR)])
    return _
  lax.fori_loop(0, nzblk, _zcopy, None)

  plsc.subcore_barrier()

  ebase0 = s * ne if fsplit else (c * NS + s) * ne

  def _chunk(j, _):
    ebase = ebase0 + j * EB
    pltpu.sync_copy(src.at[pl.ds(ebase, EB)], srcb)
    pltpu.sync_copy(dst.at[pl.ds(ebase, EB)], dstb)
    pltpu.sync_copy(w.at[pl.ds(ebase, EB)], wb)

    if fsplit:
      def _gix(g, _g):
        sv = srcb[pl.ds(g * LANES, LANES)]
        gixb[pl.ds(g * LANES, LANES)] = sv * 2 + c
        return _g
      lax.fori_loop(0, EB // LANES, _gix, None)
      pltpu.async_copy(table.at[gixb], rows, sem).wait()
    else:
      pltpu.async_copy(table.at[srcb], rows, sem).wait()

    def _scale(g, _g):
      wv = wb[pl.ds(g * LANES, LANES)]
      for l in range(LANES):
        wl = wv[l]
        for k in range(C // LANES):
          rows[g * LANES + l, pl.ds(k * LANES, LANES)] = (
              rows[g * LANES + l, pl.ds(k * LANES, LANES)] * wl)
      return _g
    lax.fori_loop(0, EB // LANES, _scale, None)

    pltpu.sync_copy(rows, acc.at[dstb], add=True)
    return _
  lax.fori_loop(0, nchunk, _chunk, None)

  plsc.subcore_barrier()

  @pl.when(s < NS - 1)
  def _():
    pltpu.sync_copy(acc.at[pl.ds(row0, big)], out.at[c, pl.ds(row0, big)])

  @pl.when(s == NS - 1)
  def _():
    base = big * (NS - 1)
    pltpu.sync_copy(acc.at[pl.ds(base, last)], out.at[c, pl.ds(base, last)])


def _make_spmm(fsplit, EB=80, ZR=80):
  mesh = plsc.VectorSubcoreMesh(core_axis_name="c", subcore_axis_name="s")
  body = functools.partial(_spmm_body, fsplit=fsplit, EB=EB, ZR=ZR)
  return pl.kernel(
      body,
      out_type=jax.ShapeDtypeStruct((NC, N_NODES, C), jnp.float32),
      mesh=mesh,
      scratch_types=[
          pltpu.VMEM_SHARED((N_NODES, C), jnp.float32),
          pltpu.VMEM((EB,), jnp.int32),
          pltpu.VMEM((EB,), jnp.int32),
          pltpu.VMEM((EB,), jnp.int32),
          pltpu.VMEM((EB,), jnp.float32),
          pltpu.VMEM((EB, C), jnp.float32),
          pltpu.VMEM((ZR, C), jnp.float32),
          pltpu.SemaphoreType.DMA,
      ],
  )


_spmm1 = _make_spmm(fsplit=True)
_spmm2 = _make_spmm(fsplit=False)


def _mlp_body(t1a_ref, t1b_ref, w1a_ref, w1b_ref, b1_ref, w2_ref, out_ref):
  h = jnp.dot(t1a_ref[...], w1a_ref[...], preferred_element_type=jnp.float32)
  h = h + jnp.dot(t1b_ref[...], w1b_ref[...],
                  preferred_element_type=jnp.float32)
  h = jnp.maximum(h + b1_ref[...], 0.0)
  out_ref[...] = jnp.dot(h, w2_ref[...], preferred_element_type=jnp.float32)


def _mlp(t1a, t1b, w1a, w1b, b1, w2t, R=400):
  n = t1a.shape[0]
  return pl.pallas_call(
      _mlp_body,
      grid=(n // R,),
      in_specs=[
          pl.BlockSpec((R, IN_FEATS // 2), lambda i: (i, 0)),
          pl.BlockSpec((R, IN_FEATS // 2), lambda i: (i, 0)),
          pl.BlockSpec((IN_FEATS // 2, H_FEATS), lambda i: (0, 0)),
          pl.BlockSpec((IN_FEATS // 2, H_FEATS), lambda i: (0, 0)),
          pl.BlockSpec((1, H_FEATS), lambda i: (0, 0)),
          pl.BlockSpec((H_FEATS, NUM_CLASSES), lambda i: (0, 0)),
      ],
      out_specs=pl.BlockSpec((R, NUM_CLASSES), lambda i: (i, 0)),
      out_shape=jax.ShapeDtypeStruct((n, NUM_CLASSES), jnp.float32),
  )(t1a, t1b, w1a, w1b, b1, w2t)


def _comb_body(p0_ref, p1_ref, b2_ref, out_ref):
  out_ref[...] = p0_ref[...] + p1_ref[...] + b2_ref[...]


def _combine(p0, p1, b2, R=1000):
  n = p0.shape[0]
  return pl.pallas_call(
      _comb_body,
      grid=(n // R,),
      in_specs=[
          pl.BlockSpec((R, NUM_CLASSES), lambda i: (i, 0)),
          pl.BlockSpec((R, NUM_CLASSES), lambda i: (i, 0)),
          pl.BlockSpec((1, NUM_CLASSES), lambda i: (0, 0)),
      ],
      out_specs=pl.BlockSpec((R, NUM_CLASSES), lambda i: (i, 0)),
      out_shape=jax.ShapeDtypeStruct((n, NUM_CLASSES), jnp.float32),
  )(p0, p1, b2)


@jax.jit
def kernel(X, edge_index, edge_weight, W1, b1, W2, b2):
  pad = E_PAD - N_EDGES
  src = jnp.pad(edge_index[1], (0, pad))
  dst = jnp.pad(edge_index[0], (0, pad))
  ew = jnp.pad(edge_weight, (0, pad))

  t1 = _spmm1(X.reshape(2 * N_NODES, IN_FEATS // 2), src, dst, ew)

  w1t = W1.T
  g = _mlp(t1[0], t1[1], w1t[: IN_FEATS // 2], w1t[IN_FEATS // 2:],
           b1.reshape(1, H_FEATS), W2.T)

  o2 = _spmm2(g, src, dst, ew)

  return _combine(o2[0], o2[1], b2.reshape(1, NUM_CLASSES))


# speedup vs baseline: 5.0052x; 1.7069x over previous
import functools

import jax
import jax.numpy as jnp
from jax import lax
from jax.experimental import pallas as pl
from jax.experimental.pallas import tpu as pltpu
from jax.experimental.pallas import tpu_sc as plsc

N_NODES = 10000
N_EDGES = 160000
EB = 128
E_PAD = 163840
IN_FEATS = 256
H_FEATS = 512
NUM_CLASSES = 128

NC = 2
NS = 16
LANES = 16
C = 128


def _spmm_body(table, gixs, dst2, w2, out, acc, ixb, dtb, wtb, rows0, rows1,
               sg0, sg1, se0, se1, se2, se3, *, fsplit, NR):
  c = lax.axis_index("c")
  s = lax.axis_index("s")
  big = 640
  last = N_NODES - big * (NS - 1)
  row0 = s * big
  br = s * NR if fsplit else (c * NS + s) * NR
  ses = (se0, se1, se2, se3)
  rbs = (rows0, rows1)
  sgs = (sg0, sg1)

  def issue_edges(jchunk, q):
    pltpu.async_copy(gixs.at[c, br + jchunk], ixb.at[q], ses[q])
    pltpu.async_copy(dst2.at[br + jchunk], dtb.at[q], ses[q])
    pltpu.async_copy(w2.at[br + jchunk], wtb.at[q], ses[q])

  def wait_edges(q):
    pltpu.make_async_copy(gixs.at[c, br], ixb.at[q], ses[q]).wait()
    pltpu.make_async_copy(dst2.at[br], dtb.at[q], ses[q]).wait()
    pltpu.make_async_copy(w2.at[br], wtb.at[q], ses[q]).wait()

  def issue_gather(q, r2):
    pltpu.async_copy(table.at[ixb.at[q]], rbs[r2], sgs[r2])

  def wait_gather(q, r2):
    pltpu.make_async_copy(table.at[ixb.at[q]], rbs[r2], sgs[r2]).wait()

  for q in range(4):
    issue_edges(q, q)

  def _zr_body(r, _):
    for k in range(C // LANES):
      rows0[r, pl.ds(k * LANES, LANES)] = jnp.zeros((LANES,), jnp.float32)
    return _
  lax.fori_loop(0, EB, _zr_body, None)

  nzblk = jnp.where(s == NS - 1, last // EB, big // EB)

  def _zcopy(j, _):
    pltpu.sync_copy(rows0, acc.at[pl.ds(row0 + j * EB, EB)])
    return _
  lax.fori_loop(0, nzblk, _zcopy, None)

  @pl.when(s == NS - 1)
  def _():
    pltpu.sync_copy(rows0.at[pl.ds(0, 16)],
                    acc.at[pl.ds(row0 + (last // EB) * EB, 16)])

  wait_edges(0)
  wait_edges(1)
  issue_gather(0, 0)
  issue_gather(1, 1)

  plsc.subcore_barrier()

  def _quad(j4, _):
    for uu in range(4):
      jj = j4 * 4 + uu
      q = uu
      r2 = uu % 2
      wait_gather(q, r2)

      def _scale(g, _g):
        wv = wtb[q, pl.ds(g * LANES, LANES)]
        for l in range(LANES):
          wl = wv[l]
          for k in range(C // LANES):
            rbs[r2][g * LANES + l, pl.ds(k * LANES, LANES)] = (
                rbs[r2][g * LANES + l, pl.ds(k * LANES, LANES)] * wl)
        return _g
      lax.fori_loop(0, EB // LANES, _scale, None)

      pltpu.sync_copy(rbs[r2], acc.at[dtb.at[q]], add=True)

      @pl.when(jj + 4 < NR)
      def _():
        issue_edges(jj + 4, q)

      @pl.when(jj + 2 < NR)
      def _():
        wait_edges((q + 2) % 4)
        issue_gather((q + 2) % 4, r2)
    return _
  lax.fori_loop(0, NR // 4, _quad, None)

  plsc.subcore_barrier()

  @pl.when(s < NS - 1)
  def _():
    pltpu.sync_copy(acc.at[pl.ds(row0, big)], out.at[c, pl.ds(row0, big)])

  @pl.when(s == NS - 1)
  def _():
    base = big * (NS - 1)
    pltpu.sync_copy(acc.at[pl.ds(base, last)], out.at[c, pl.ds(base, last)])


def _make_spmm(fsplit):
  nwork = NS if fsplit else NC * NS
  NR = E_PAD // EB // nwork
  mesh = plsc.VectorSubcoreMesh(core_axis_name="c", subcore_axis_name="s")
  body = functools.partial(_spmm_body, fsplit=fsplit, NR=NR)
  return pl.kernel(
      body,
      out_type=jax.ShapeDtypeStruct((NC, N_NODES, C), jnp.float32),
      mesh=mesh,
      scratch_types=[
          pltpu.VMEM_SHARED((N_NODES, C), jnp.float32),
          pltpu.VMEM((4, EB), jnp.int32),
          pltpu.VMEM((4, EB), jnp.int32),
          pltpu.VMEM((4, EB), jnp.float32),
          pltpu.VMEM((EB, C), jnp.float32),
          pltpu.VMEM((EB, C), jnp.float32),
          pltpu.SemaphoreType.DMA,
          pltpu.SemaphoreType.DMA,
          pltpu.SemaphoreType.DMA,
          pltpu.SemaphoreType.DMA,
          pltpu.SemaphoreType.DMA,
          pltpu.SemaphoreType.DMA,
      ],
  )


_spmm1 = _make_spmm(fsplit=True)
_spmm2 = _make_spmm(fsplit=False)


def _mlp_body(t1a_ref, t1b_ref, w1a_ref, w1b_ref, b1_ref, w2_ref, out_ref):
  h = jnp.dot(t1a_ref[...], w1a_ref[...], preferred_element_type=jnp.float32)
  h = h + jnp.dot(t1b_ref[...], w1b_ref[...],
                  preferred_element_type=jnp.float32)
  h = jnp.maximum(h + b1_ref[...], 0.0)
  out_ref[...] = jnp.dot(h, w2_ref[...], preferred_element_type=jnp.float32)


def _mlp(t1a, t1b, w1a, w1b, b1, w2t, R=400):
  n = t1a.shape[0]
  return pl.pallas_call(
      _mlp_body,
      grid=(n // R,),
      in_specs=[
          pl.BlockSpec((R, IN_FEATS // 2), lambda i: (i, 0)),
          pl.BlockSpec((R, IN_FEATS // 2), lambda i: (i, 0)),
          pl.BlockSpec((IN_FEATS // 2, H_FEATS), lambda i: (0, 0)),
          pl.BlockSpec((IN_FEATS // 2, H_FEATS), lambda i: (0, 0)),
          pl.BlockSpec((1, H_FEATS), lambda i: (0, 0)),
          pl.BlockSpec((H_FEATS, NUM_CLASSES), lambda i: (0, 0)),
      ],
      out_specs=pl.BlockSpec((R, NUM_CLASSES), lambda i: (i, 0)),
      out_shape=jax.ShapeDtypeStruct((n, NUM_CLASSES), jnp.float32),
  )(t1a, t1b, w1a, w1b, b1, w2t)


def _comb_body(p0_ref, p1_ref, b2_ref, out_ref):
  out_ref[...] = p0_ref[...] + p1_ref[...] + b2_ref[...]


def _combine(p0, p1, b2, R=1000):
  n = p0.shape[0]
  return pl.pallas_call(
      _comb_body,
      grid=(n // R,),
      in_specs=[
          pl.BlockSpec((R, NUM_CLASSES), lambda i: (i, 0)),
          pl.BlockSpec((R, NUM_CLASSES), lambda i: (i, 0)),
          pl.BlockSpec((1, NUM_CLASSES), lambda i: (0, 0)),
      ],
      out_specs=pl.BlockSpec((R, NUM_CLASSES), lambda i: (i, 0)),
      out_shape=jax.ShapeDtypeStruct((n, NUM_CLASSES), jnp.float32),
  )(p0, p1, b2)


@jax.jit
def kernel(X, edge_index, edge_weight, W1, b1, W2, b2):
  pad = E_PAD - N_EDGES
  nrow = E_PAD // EB
  src = jnp.pad(edge_index[1], (0, pad)).reshape(nrow, EB)
  dst = jnp.pad(edge_index[0], (0, pad)).reshape(nrow, EB)
  ew = jnp.pad(edge_weight, (0, pad)).reshape(nrow, EB)

  gix1 = jnp.stack([2 * src, 2 * src + 1])
  t1 = _spmm1(X.reshape(2 * N_NODES, IN_FEATS // 2), gix1, dst, ew)

  w1t = W1.T
  g = _mlp(t1[0], t1[1], w1t[: IN_FEATS // 2], w1t[IN_FEATS // 2:],
           b1.reshape(1, H_FEATS), W2.T)

  gix2 = jnp.stack([src, src])
  o2 = _spmm2(g, gix2, dst, ew)

  return _combine(o2[0], o2[1], b2.reshape(1, NUM_CLASSES))
